# Initial kernel scaffold; baseline (speedup 1.0000x reference)
import jax
import jax.numpy as jnp
from jax.experimental import pallas as pl


def kernel(node_id, img_h, txt_h, table):
    raise NotImplementedError("write your pallas kernel here")


# speedup vs baseline: 1.3203x; 1.3203x over previous
import functools

import jax
import jax.numpy as jnp
from jax import lax
from jax.experimental import pallas as pl
from jax.experimental.pallas import tpu as pltpu
from jax.experimental.pallas import tpu_sc as plsc

NC, NS = 2, 16
NW = NC * NS
C = 80
T = 40
PER_W = C * T
N_PAD = NW * PER_W


def _emb_body(idx_hbm, table_hbm, out_hbm, idx_v, rows_v, sem):
    n_valid_chunks = out_hbm.shape[0] // C
    w = lax.axis_index("s") * NC + lax.axis_index("c")
    base = w * PER_W
    pltpu.sync_copy(idx_hbm.at[pl.ds(base, PER_W)], idx_v)

    def body(t, carry):
        chunk = w * T + t

        @pl.when(chunk < n_valid_chunks)
        def _():
            pltpu.async_copy(
                table_hbm.at[idx_v.at[pl.ds(t * C, C)]], rows_v, sem
            ).wait()
            pltpu.sync_copy(rows_v, out_hbm.at[pl.ds(chunk * C, C)])

        return carry

    lax.fori_loop(0, T, body, 0)


def kernel(node_id, img_h, txt_h, table):
    n = node_id.shape[0]
    idx = node_id.astype(jnp.int32)
    idx = jnp.concatenate([idx, jnp.zeros((N_PAD - n,), jnp.int32)])
    mesh = plsc.VectorSubcoreMesh(core_axis_name="c", subcore_axis_name="s")
    f = pl.kernel(
        _emb_body,
        out_type=jax.ShapeDtypeStruct((n, table.shape[1]), table.dtype),
        mesh=mesh,
        scratch_types=[
            pltpu.VMEM((PER_W,), jnp.int32),
            pltpu.VMEM((C, table.shape[1]), jnp.float32),
            pltpu.SemaphoreType.DMA,
        ],
    )
    return f(idx, table)


# speedup vs baseline: 2.0208x; 1.5305x over previous
import jax
import jax.numpy as jnp
from jax import lax
from jax.experimental import pallas as pl
from jax.experimental.pallas import tpu as pltpu
from jax.experimental.pallas import tpu_sc as plsc

NC, NS = 2, 16
NW = NC * NS
C = 80
T = 40
PER_W = C * T
N_PAD = NW * PER_W
G = 4
NG = T // G


def _emb_body(idx_hbm, table_hbm, out_hbm, idx_v, buf0, buf1, gsem0, gsem1):
    n_valid = out_hbm.shape[0] // C
    w = lax.axis_index("s") * NC + lax.axis_index("c")
    pltpu.sync_copy(idx_hbm.at[pl.ds(w * PER_W, PER_W)], idx_v)

    bufs = (buf0, buf1)
    gsems = (gsem0, gsem1)

    def fire(g, b):
        for j in range(G):
            tt = g * G + j
            chunk = w * T + tt

            @pl.when((tt < T) & (chunk < n_valid))
            def _():
                pltpu.async_copy(
                    table_hbm.at[idx_v.at[pl.ds(tt * C, C)]],
                    bufs[b].at[pl.ds(j * C, C)],
                    gsems[b],
                )

    def drain_write(g, b):
        for j in range(G):
            tt = g * G + j
            chunk = w * T + tt

            @pl.when((tt < T) & (chunk < n_valid))
            def _():
                pltpu.make_async_copy(
                    table_hbm.at[idx_v.at[pl.ds(tt * C, C)]],
                    bufs[b].at[pl.ds(j * C, C)],
                    gsems[b],
                ).wait()
                pltpu.sync_copy(
                    bufs[b].at[pl.ds(j * C, C)],
                    out_hbm.at[pl.ds(chunk * C, C)],
                )

    fire(0, 0)
    fire(1, 1)

    def body(v, carry):
        g0 = 2 * v
        drain_write(g0, 0)
        fire(g0 + 2, 0)
        drain_write(g0 + 1, 1)
        fire(g0 + 3, 1)
        return carry

    lax.fori_loop(0, NG // 2, body, 0)


def kernel(node_id, img_h, txt_h, table):
    n = node_id.shape[0]
    idx = node_id.astype(jnp.int32)
    idx = jnp.concatenate([idx, jnp.zeros((N_PAD - n,), jnp.int32)])
    mesh = plsc.VectorSubcoreMesh(core_axis_name="c", subcore_axis_name="s")
    f = pl.kernel(
        _emb_body,
        out_type=jax.ShapeDtypeStruct((n, table.shape[1]), table.dtype),
        mesh=mesh,
        scratch_types=[
            pltpu.VMEM((PER_W,), jnp.int32),
            pltpu.VMEM((G * C, table.shape[1]), jnp.float32),
            pltpu.VMEM((G * C, table.shape[1]), jnp.float32),
            pltpu.SemaphoreType.DMA,
            pltpu.SemaphoreType.DMA,
        ],
    )
    return f(idx, table)


# speedup vs baseline: 2.0482x; 1.0136x over previous
import jax
import jax.numpy as jnp
from jax import lax
from jax.experimental import pallas as pl
from jax.experimental.pallas import tpu as pltpu
from jax.experimental.pallas import tpu_sc as plsc

NC, NS = 2, 16
NW = NC * NS
C = 80
T = 40
PER_W = C * T
N_PAD = NW * PER_W
G = 4
NG = T // G


def _emb_body(idx_hbm, table_hbm, out_hbm, idx_v, buf0, buf1, gsem0, gsem1):
    n_valid = out_hbm.shape[0] // C
    n_idx = idx_hbm.shape[0]
    tail_w = (n_idx - 1) // PER_W
    tail_len = n_idx - tail_w * PER_W
    w = lax.axis_index("s") * NC + lax.axis_index("c")

    @pl.when(w < tail_w)
    def _():
        pltpu.sync_copy(idx_hbm.at[pl.ds(w * PER_W, PER_W)], idx_v)

    @pl.when(w == tail_w)
    def _():
        pltpu.sync_copy(
            idx_hbm.at[pl.ds(tail_w * PER_W, tail_len)],
            idx_v.at[pl.ds(0, tail_len)],
        )

    bufs = (buf0, buf1)
    gsems = (gsem0, gsem1)

    def fire(g, b):
        for j in range(G):
            tt = g * G + j
            chunk = w * T + tt

            @pl.when((tt < T) & (chunk < n_valid))
            def _():
                pltpu.async_copy(
                    table_hbm.at[idx_v.at[pl.ds(tt * C, C)]],
                    bufs[b].at[pl.ds(j * C, C)],
                    gsems[b],
                )

    def drain_write(g, b):
        for j in range(G):
            tt = g * G + j
            chunk = w * T + tt

            @pl.when((tt < T) & (chunk < n_valid))
            def _():
                pltpu.make_async_copy(
                    table_hbm.at[idx_v.at[pl.ds(tt * C, C)]],
                    bufs[b].at[pl.ds(j * C, C)],
                    gsems[b],
                ).wait()
                pltpu.sync_copy(
                    bufs[b].at[pl.ds(j * C, C)],
                    out_hbm.at[pl.ds(chunk * C, C)],
                )

    fire(0, 0)
    fire(1, 1)

    def body(v, carry):
        g0 = 2 * v
        drain_write(g0, 0)
        fire(g0 + 2, 0)
        drain_write(g0 + 1, 1)
        fire(g0 + 3, 1)
        return carry

    lax.fori_loop(0, NG // 2, body, 0)


def kernel(node_id, img_h, txt_h, table):
    n = node_id.shape[0]
    idx = node_id.astype(jnp.int32)
    mesh = plsc.VectorSubcoreMesh(core_axis_name="c", subcore_axis_name="s")
    f = pl.kernel(
        _emb_body,
        out_type=jax.ShapeDtypeStruct((n, table.shape[1]), table.dtype),
        mesh=mesh,
        scratch_types=[
            pltpu.VMEM((PER_W,), jnp.int32),
            pltpu.VMEM((G * C, table.shape[1]), jnp.float32),
            pltpu.VMEM((G * C, table.shape[1]), jnp.float32),
            pltpu.SemaphoreType.DMA,
            pltpu.SemaphoreType.DMA,
        ],
    )
    return f(idx, table)
